# baseline (speedup 1.0000x reference)
import jax
import jax.numpy as jnp
from jax import lax
from jax.experimental import pallas as pl
from jax.experimental.pallas import tpu as pltpu
from jax.experimental.pallas import tpu_sc as plsc

_VSZ = 1000001
_DSZ = 64
_B = 4096
_L = 200

_SROWS = 1000000
_CH = 320
_NCHUNK = _SROWS // _CH
_NKMAX = -(-_NCHUNK // 32)

_NW = 32
_BPW = _B // _NW
_TPW = _BPW * _L
_GCHUNK = 128
_NCH = _TPW // _GCHUNK

_DN = lax.GatherDimensionNumbers(offset_dims=(), collapsed_slice_dims=(0,),
                                 start_index_map=(0,))


def _lane_sum(p, perms):
    for perm in perms:
        p = p + lax.gather(p, perm, _DN, slice_sizes=(1,),
                           mode=lax.GatherScatterMode.PROMISE_IN_BOUNDS)
    return p


def _sc_scan_body(lut_hbm, w_hbm, pd_hbm, wv, buf_a, buf_b, out_v,
                  sem_a, sem_b, sem_o):
    c = lax.axis_index("c")
    s = lax.axis_index("s")
    wid = s * 2 + c
    nk = jnp.where(wid < _NCHUNK % 32, _NKMAX, _NKMAX - 1)

    pltpu.sync_copy(w_hbm, wv)
    wd = [wv[1, pl.ds(q * 16, 16)] - wv[0, pl.ds(q * 16, 16)]
          for q in range(4)]
    lane = lax.iota(jnp.int32, 16)
    perms = [jnp.reshape(lane ^ sh, (16, 1)) for sh in (8, 4, 2, 1)]
    lmask = [lane == uu for uu in range(16)]

    bufs = (buf_a, buf_b)
    sems = (sem_a, sem_b)

    def row0(k):
        return (wid + 32 * k) * _CH

    def issue(k, u):
        pltpu.async_copy(lut_hbm.at[pl.ds(row0(k), _CH)], bufs[u], sems[u])

    def wait(u):
        pltpu.make_async_copy(lut_hbm.at[pl.ds(0, _CH)], bufs[u],
                              sems[u]).wait()

    def process(k, u):
        buf = bufs[u]

        def group(i, carry):
            res = jnp.zeros((16,), jnp.float32)
            for uu in range(16):
                r = i * 16 + uu
                p = (buf[r, pl.ds(0, 16)] * wd[0]
                     + buf[r, pl.ds(16, 16)] * wd[1]
                     + buf[r, pl.ds(32, 16)] * wd[2]
                     + buf[r, pl.ds(48, 16)] * wd[3])
                p = _lane_sum(p, perms)
                res = jnp.where(lmask[uu], p, res)
            out_v[pl.ds(k * _CH + i * 16, 16)] = res
            return carry

        lax.fori_loop(0, _CH // 16, group, 0, unroll=2)
        pltpu.async_copy(out_v.at[pl.ds(k * _CH, _CH)],
                         pd_hbm.at[pl.ds(row0(k), _CH)], sem_o)

    issue(0, 0)
    issue(1, 1)

    def chunk_pair(g, carry):
        for u in range(2):
            k = g * 2 + u
            wait(u)
            process(k, u)

            @pl.when(k + 2 < nk)
            def _():
                issue(k + 2, u)

        return carry

    lax.fori_loop(0, nk // 2, chunk_pair, 0, unroll=False)

    @pl.when(nk % 2 == 1)
    def _tail():
        wait(0)
        process(nk - 1, 0)

    def drain(k, carry):
        pltpu.make_async_copy(out_v.at[pl.ds(0, _CH)],
                              pd_hbm.at[pl.ds(0, _CH)], sem_o).wait()
        return carry

    lax.fori_loop(0, nk, drain, 0, unroll=False)


def _sc_scan(lut_weight, out_weight):
    mesh = plsc.VectorSubcoreMesh(core_axis_name="c", subcore_axis_name="s")
    run = pl.kernel(
        _sc_scan_body,
        out_type=jax.ShapeDtypeStruct((_SROWS,), jnp.float32),
        mesh=mesh,
        scratch_types=[
            pltpu.VMEM((2, _DSZ), jnp.float32),
            pltpu.VMEM((_CH, _DSZ), jnp.float32),
            pltpu.VMEM((_CH, _DSZ), jnp.float32),
            pltpu.VMEM((_NKMAX * _CH,), jnp.float32),
            pltpu.SemaphoreType.DMA,
            pltpu.SemaphoreType.DMA,
            pltpu.SemaphoreType.DMA,
        ],
    )
    return run(lut_weight, out_weight)


def _sc_body(pd_hbm, ids_hbm, d_hbm, idx_v, vals_v, out_v, sem):
    c = lax.axis_index("c")
    s = lax.axis_index("s")
    wid = s * 2 + c

    pltpu.sync_copy(ids_hbm.at[wid], idx_v)

    nacc = _BPW // 16

    def issue(j, carry):
        pltpu.async_copy(
            pd_hbm.at[idx_v.at[j]],
            vals_v.at[pl.ds(j * _GCHUNK, _GCHUNK)],
            sem)
        return carry

    lax.fori_loop(0, _NCH, issue, 0, unroll=False)
    pltpu.make_async_copy(pd_hbm.at[pl.ds(0, _TPW)], vals_v, sem).wait()

    def acc_chunk(j, accs):
        base = j * _GCHUNK
        return tuple(
            accs[r] + vals_v[pl.ds(base + r * 16, 16)]
            for r in range(nacc))

    accs = lax.fori_loop(
        0, _NCH, acc_chunk,
        tuple(jnp.zeros((16,), jnp.float32) for _ in range(nacc)),
        unroll=False)

    for r in range(nacc):
        out_v[pl.ds(r * 16, 16)] = accs[r] * (1.0 / _L)

    pltpu.sync_copy(out_v, d_hbm.at[pl.ds(wid * _BPW, _BPW)])


def _sc_gather_mean(pd_flat, ids3):
    mesh = plsc.VectorSubcoreMesh(core_axis_name="c", subcore_axis_name="s")
    run = pl.kernel(
        _sc_body,
        out_type=jax.ShapeDtypeStruct((_B,), jnp.float32),
        mesh=mesh,
        scratch_types=[
            pltpu.VMEM((_NCH, _GCHUNK), jnp.int32),
            pltpu.VMEM((_TPW,), jnp.float32),
            pltpu.VMEM((_BPW,), jnp.float32),
            pltpu.SemaphoreType.DMA,
        ],
    )
    return run(pd_flat, ids3)


def _head_body(d_ref, b_ref, o0_ref, o1_ref):
    delta = d_ref[...] + (b_ref[1] - b_ref[0])
    sp = jnp.maximum(delta, 0.0) + jnp.log1p(jnp.exp(-jnp.abs(delta)))
    o0_ref[...] = -sp
    o1_ref[...] = delta - sp


def _head(d2, out_bias):
    return pl.pallas_call(
        _head_body,
        in_specs=[pl.BlockSpec((_NW, _BPW), lambda: (0, 0)),
                  pl.BlockSpec(memory_space=pltpu.SMEM)],
        out_specs=[pl.BlockSpec((_NW, _BPW), lambda: (0, 0))] * 2,
        out_shape=[jax.ShapeDtypeStruct((_NW, _BPW), jnp.float32)] * 2,
    )(d2, out_bias)


def kernel(input, lut_weight, out_weight, out_bias):
    ids = input.astype(jnp.int32)
    pd = _sc_scan(lut_weight, out_weight)
    ids3 = ids.reshape(_NW, _BPW, _L).transpose(0, 2, 1)
    delta = _sc_gather_mean(pd, ids3)
    o0, o1 = _head(delta.reshape(_NW, _BPW), out_bias)
    return jnp.stack([o0.reshape(_B), o1.reshape(_B)], axis=-1)


# speedup vs baseline: 1.0050x; 1.0050x over previous
import jax
import jax.numpy as jnp
from jax import lax
from jax.experimental import pallas as pl
from jax.experimental.pallas import tpu as pltpu
from jax.experimental.pallas import tpu_sc as plsc

_VSZ = 1000001
_DSZ = 64
_B = 4096
_L = 200

_SROWS = 1000000
_CH = 320
_NCHUNK = _SROWS // _CH
_NKMAX = -(-_NCHUNK // 32)

_NW = 32
_BPW = _B // _NW
_TPW = _BPW * _L
_GCHUNK = 128
_NCH = _TPW // _GCHUNK

_DN = lax.GatherDimensionNumbers(offset_dims=(), collapsed_slice_dims=(0,),
                                 start_index_map=(0,))


def _lane_sum(p, perms):
    for perm in perms:
        p = p + lax.gather(p, perm, _DN, slice_sizes=(1,),
                           mode=lax.GatherScatterMode.PROMISE_IN_BOUNDS)
    return p


def _sc_scan_body(lut_hbm, w_hbm, pd_hbm, wv, buf_a, buf_b, out_v,
                  sem_a, sem_b, sem_o):
    c = lax.axis_index("c")
    s = lax.axis_index("s")
    wid = s * 2 + c
    nk = jnp.where(wid < _NCHUNK % 32, _NKMAX, _NKMAX - 1)
    base_row = (_NKMAX * wid - jnp.maximum(0, wid - _NCHUNK % 32)) * _CH

    pltpu.sync_copy(w_hbm, wv)
    wd = [wv[1, pl.ds(q * 16, 16)] - wv[0, pl.ds(q * 16, 16)]
          for q in range(4)]
    lane = lax.iota(jnp.int32, 16)
    perms = [jnp.reshape(lane ^ sh, (16, 1)) for sh in (8, 4, 2, 1)]
    lmask = [lane == uu for uu in range(16)]

    bufs = (buf_a, buf_b)
    sems = (sem_a, sem_b)

    def row0(k):
        return base_row + k * _CH

    def issue(k, u):
        pltpu.async_copy(lut_hbm.at[pl.ds(row0(k), _CH)], bufs[u], sems[u])

    def wait(u):
        pltpu.make_async_copy(lut_hbm.at[pl.ds(0, _CH)], bufs[u],
                              sems[u]).wait()

    def process(k, u):
        buf = bufs[u]

        def group(i, carry):
            res = jnp.zeros((16,), jnp.float32)
            for uu in range(16):
                r = i * 16 + uu
                p = (buf[r, pl.ds(0, 16)] * wd[0]
                     + buf[r, pl.ds(16, 16)] * wd[1]
                     + buf[r, pl.ds(32, 16)] * wd[2]
                     + buf[r, pl.ds(48, 16)] * wd[3])
                p = _lane_sum(p, perms)
                res = jnp.where(lmask[uu], p, res)
            out_v[pl.ds(k * _CH + i * 16, 16)] = res
            return carry

        lax.fori_loop(0, _CH // 16, group, 0, unroll=2)
        pltpu.async_copy(out_v.at[pl.ds(k * _CH, _CH)],
                         pd_hbm.at[pl.ds(row0(k), _CH)], sem_o)

    issue(0, 0)
    issue(1, 1)

    def chunk_pair(g, carry):
        for u in range(2):
            k = g * 2 + u
            wait(u)
            process(k, u)

            @pl.when(k + 2 < nk)
            def _():
                issue(k + 2, u)

        return carry

    lax.fori_loop(0, nk // 2, chunk_pair, 0, unroll=False)

    @pl.when(nk % 2 == 1)
    def _tail():
        wait(0)
        process(nk - 1, 0)

    def drain(k, carry):
        pltpu.make_async_copy(out_v.at[pl.ds(0, _CH)],
                              pd_hbm.at[pl.ds(0, _CH)], sem_o).wait()
        return carry

    lax.fori_loop(0, nk, drain, 0, unroll=False)


def _sc_scan(lut_weight, out_weight):
    mesh = plsc.VectorSubcoreMesh(core_axis_name="c", subcore_axis_name="s")
    run = pl.kernel(
        _sc_scan_body,
        out_type=jax.ShapeDtypeStruct((_SROWS,), jnp.float32),
        mesh=mesh,
        scratch_types=[
            pltpu.VMEM((2, _DSZ), jnp.float32),
            pltpu.VMEM((_CH, _DSZ), jnp.float32),
            pltpu.VMEM((_CH, _DSZ), jnp.float32),
            pltpu.VMEM((_NKMAX * _CH,), jnp.float32),
            pltpu.SemaphoreType.DMA,
            pltpu.SemaphoreType.DMA,
            pltpu.SemaphoreType.DMA,
        ],
    )
    return run(lut_weight, out_weight)


def _sc_body(pd_hbm, ids_hbm, d_hbm, idx_v, vals_v, out_v, sem):
    c = lax.axis_index("c")
    s = lax.axis_index("s")
    wid = s * 2 + c

    pltpu.sync_copy(ids_hbm.at[wid], idx_v)

    nacc = _BPW // 16

    def issue(j, carry):
        pltpu.async_copy(
            pd_hbm.at[idx_v.at[j]],
            vals_v.at[pl.ds(j * _GCHUNK, _GCHUNK)],
            sem)
        return carry

    lax.fori_loop(0, _NCH, issue, 0, unroll=False)
    pltpu.make_async_copy(pd_hbm.at[pl.ds(0, _TPW)], vals_v, sem).wait()

    def acc_chunk(j, accs):
        base = j * _GCHUNK
        return tuple(
            accs[r] + vals_v[pl.ds(base + r * 16, 16)]
            for r in range(nacc))

    accs = lax.fori_loop(
        0, _NCH, acc_chunk,
        tuple(jnp.zeros((16,), jnp.float32) for _ in range(nacc)),
        unroll=False)

    for r in range(nacc):
        out_v[pl.ds(r * 16, 16)] = accs[r] * (1.0 / _L)

    pltpu.sync_copy(out_v, d_hbm.at[pl.ds(wid * _BPW, _BPW)])


def _sc_gather_mean(pd_flat, ids3):
    mesh = plsc.VectorSubcoreMesh(core_axis_name="c", subcore_axis_name="s")
    run = pl.kernel(
        _sc_body,
        out_type=jax.ShapeDtypeStruct((_B,), jnp.float32),
        mesh=mesh,
        scratch_types=[
            pltpu.VMEM((_NCH, _GCHUNK), jnp.int32),
            pltpu.VMEM((_TPW,), jnp.float32),
            pltpu.VMEM((_BPW,), jnp.float32),
            pltpu.SemaphoreType.DMA,
        ],
    )
    return run(pd_flat, ids3)


def _head_body(d_ref, b_ref, o0_ref, o1_ref):
    delta = d_ref[...] + (b_ref[1] - b_ref[0])
    sp = jnp.maximum(delta, 0.0) + jnp.log1p(jnp.exp(-jnp.abs(delta)))
    o0_ref[...] = -sp
    o1_ref[...] = delta - sp


def _head(d2, out_bias):
    return pl.pallas_call(
        _head_body,
        in_specs=[pl.BlockSpec((_NW, _BPW), lambda: (0, 0)),
                  pl.BlockSpec(memory_space=pltpu.SMEM)],
        out_specs=[pl.BlockSpec((_NW, _BPW), lambda: (0, 0))] * 2,
        out_shape=[jax.ShapeDtypeStruct((_NW, _BPW), jnp.float32)] * 2,
    )(d2, out_bias)


def kernel(input, lut_weight, out_weight, out_bias):
    ids = input.astype(jnp.int32)
    pd = _sc_scan(lut_weight, out_weight)
    ids3 = ids.reshape(_NW, _BPW, _L).transpose(0, 2, 1)
    delta = _sc_gather_mean(pd, ids3)
    o0, o1 = _head(delta.reshape(_NW, _BPW), out_bias)
    return jnp.stack([o0.reshape(_B), o1.reshape(_B)], axis=-1)


# speedup vs baseline: 1.1066x; 1.1011x over previous
import functools

import jax
import jax.numpy as jnp
from jax import lax
from jax.experimental import pallas as pl
from jax.experimental.pallas import tpu as pltpu
from jax.experimental.pallas import tpu_sc as plsc

_VSZ = 1000001
_DSZ = 64
_B = 4096
_L = 200

_ROW_BLK = 16384
_NBLK = -(-_VSZ // _ROW_BLK)
_NW = 32
_BPW = _B // _NW
_TPW = _BPW * _L
_GCHUNK = 128
_NCH = _TPW // _GCHUNK
_FIRE = 8


def _proj_body(lut_ref, w_ref, b_ref, pd_ref):
    w = w_ref[...]
    wd = w[1:2, :] - w[0:1, :]
    bd = b_ref[1] - b_ref[0]
    x = lut_ref[...]
    pd = lax.dot_general(wd, x, (((1,), (1,)), ((), ())),
                         preferred_element_type=jnp.float32)
    pd_ref[...] = (pd + bd).reshape(1, 1, _ROW_BLK)


def _project_table(lut_weight, out_weight, out_bias):
    return pl.pallas_call(
        _proj_body,
        grid=(_NBLK,),
        in_specs=[
            pl.BlockSpec((_ROW_BLK, _DSZ), lambda i: (i, 0)),
            pl.BlockSpec((2, _DSZ), lambda i: (0, 0)),
            pl.BlockSpec(memory_space=pltpu.SMEM),
        ],
        out_specs=pl.BlockSpec((1, 1, _ROW_BLK), lambda i: (i, 0, 0)),
        out_shape=jax.ShapeDtypeStruct((_NBLK, 1, _ROW_BLK), jnp.float32),
    )(lut_weight, out_weight, out_bias)


def _sc_body(pd_hbm, ids_hbm, d_hbm, idx_v, vals_v, out_v, sem):
    c = lax.axis_index("c")
    s = lax.axis_index("s")
    wid = s * 2 + c

    pltpu.sync_copy(ids_hbm.at[wid], idx_v)

    nacc = _BPW // 16

    def issue(j, carry):
        pltpu.async_copy(
            pd_hbm.at[idx_v.at[j]],
            vals_v.at[pl.ds(j * _GCHUNK, _GCHUNK)],
            sem)
        return carry

    lax.fori_loop(0, _NCH, issue, 0, unroll=False)

    pltpu.make_async_copy(pd_hbm.at[pl.ds(0, _TPW)], vals_v, sem).wait()

    def acc_chunk(j, accs):
        base = j * _GCHUNK
        return tuple(
            accs[r] + vals_v[pl.ds(base + r * 16, 16)]
            for r in range(nacc))

    accs = lax.fori_loop(
        0, _NCH, acc_chunk,
        tuple(jnp.zeros((16,), jnp.float32) for _ in range(nacc)),
        unroll=False)

    for r in range(nacc):
        out_v[pl.ds(r * 16, 16)] = accs[r] * (1.0 / _L)

    pltpu.sync_copy(out_v, d_hbm.at[pl.ds(wid * _BPW, _BPW)])


def _sc_gather_mean(pd_flat, ids3):
    mesh = plsc.VectorSubcoreMesh(core_axis_name="c", subcore_axis_name="s")
    run = pl.kernel(
        _sc_body,
        out_type=jax.ShapeDtypeStruct((_B,), jnp.float32),
        mesh=mesh,
        scratch_types=[
            pltpu.VMEM((_NCH, _GCHUNK), jnp.int32),
            pltpu.VMEM((_TPW,), jnp.float32),
            pltpu.VMEM((_BPW,), jnp.float32),
            pltpu.SemaphoreType.DMA,
        ],
    )
    return run(pd_flat, ids3)


def _head_body(d_ref, o0_ref, o1_ref):
    delta = d_ref[...]
    sp = jnp.maximum(delta, 0.0) + jnp.log1p(jnp.exp(-jnp.abs(delta)))
    o0_ref[...] = -sp
    o1_ref[...] = delta - sp


def _head(d2):
    return pl.pallas_call(
        _head_body,
        in_specs=[pl.BlockSpec((_NW, _BPW), lambda: (0, 0))],
        out_specs=[pl.BlockSpec((_NW, _BPW), lambda: (0, 0))] * 2,
        out_shape=[jax.ShapeDtypeStruct((_NW, _BPW), jnp.float32)] * 2,
    )(d2)


def kernel(input, lut_weight, out_weight, out_bias):
    ids = input.astype(jnp.int32)
    pd = _project_table(lut_weight, out_weight, out_bias).reshape(-1)
    ids3 = ids.reshape(_NW, _BPW, _L).transpose(0, 2, 1)
    delta = _sc_gather_mean(pd, ids3)
    o0, o1 = _head(delta.reshape(_NW, _BPW))
    return jnp.stack([o0.reshape(_B), o1.reshape(_B)], axis=-1)


# speedup vs baseline: 1.1091x; 1.0022x over previous
import functools

import jax
import jax.numpy as jnp
from jax import lax
from jax.experimental import pallas as pl
from jax.experimental.pallas import tpu as pltpu
from jax.experimental.pallas import tpu_sc as plsc

_VSZ = 1000001
_DSZ = 64
_B = 4096
_L = 200

_ROW_BLK = 32768
_NBLK = -(-_VSZ // _ROW_BLK)
_NW = 32
_BPW = _B // _NW
_TPW = _BPW * _L
_GCHUNK = 128
_NCH = _TPW // _GCHUNK
_FIRE = 8


def _proj_body(lut_ref, w_ref, b_ref, pd_ref):
    w = w_ref[...]
    wd = w[1:2, :] - w[0:1, :]
    bd = b_ref[1] - b_ref[0]
    x = lut_ref[...]
    pd = lax.dot_general(wd, x, (((1,), (1,)), ((), ())),
                         preferred_element_type=jnp.float32)
    pd_ref[...] = (pd + bd).reshape(1, 1, _ROW_BLK)


def _project_table(lut_weight, out_weight, out_bias):
    return pl.pallas_call(
        _proj_body,
        grid=(_NBLK,),
        in_specs=[
            pl.BlockSpec((_ROW_BLK, _DSZ), lambda i: (i, 0)),
            pl.BlockSpec((2, _DSZ), lambda i: (0, 0)),
            pl.BlockSpec(memory_space=pltpu.SMEM),
        ],
        out_specs=pl.BlockSpec((1, 1, _ROW_BLK), lambda i: (i, 0, 0)),
        out_shape=jax.ShapeDtypeStruct((_NBLK, 1, _ROW_BLK), jnp.float32),
    )(lut_weight, out_weight, out_bias)


def _sc_body(pd_hbm, ids_hbm, d_hbm, idx_v, vals_v, out_v, sem):
    c = lax.axis_index("c")
    s = lax.axis_index("s")
    wid = s * 2 + c

    pltpu.sync_copy(ids_hbm.at[wid], idx_v)

    nacc = _BPW // 16

    def issue(j, carry):
        pltpu.async_copy(
            pd_hbm.at[idx_v.at[j]],
            vals_v.at[pl.ds(j * _GCHUNK, _GCHUNK)],
            sem)
        return carry

    lax.fori_loop(0, _NCH, issue, 0, unroll=False)

    pltpu.make_async_copy(pd_hbm.at[pl.ds(0, _TPW)], vals_v, sem).wait()

    def acc_chunk(j, accs):
        base = j * _GCHUNK
        return tuple(
            accs[r] + vals_v[pl.ds(base + r * 16, 16)]
            for r in range(nacc))

    accs = lax.fori_loop(
        0, _NCH, acc_chunk,
        tuple(jnp.zeros((16,), jnp.float32) for _ in range(nacc)),
        unroll=False)

    for r in range(nacc):
        out_v[pl.ds(r * 16, 16)] = accs[r] * (1.0 / _L)

    pltpu.sync_copy(out_v, d_hbm.at[pl.ds(wid * _BPW, _BPW)])


def _sc_gather_mean(pd_flat, ids3):
    mesh = plsc.VectorSubcoreMesh(core_axis_name="c", subcore_axis_name="s")
    run = pl.kernel(
        _sc_body,
        out_type=jax.ShapeDtypeStruct((_B,), jnp.float32),
        mesh=mesh,
        scratch_types=[
            pltpu.VMEM((_NCH, _GCHUNK), jnp.int32),
            pltpu.VMEM((_TPW,), jnp.float32),
            pltpu.VMEM((_BPW,), jnp.float32),
            pltpu.SemaphoreType.DMA,
        ],
    )
    return run(pd_flat, ids3)


def _head_body(d_ref, o0_ref, o1_ref):
    delta = d_ref[...]
    sp = jnp.maximum(delta, 0.0) + jnp.log1p(jnp.exp(-jnp.abs(delta)))
    o0_ref[...] = -sp
    o1_ref[...] = delta - sp


def _head(d2):
    return pl.pallas_call(
        _head_body,
        in_specs=[pl.BlockSpec((_NW, _BPW), lambda: (0, 0))],
        out_specs=[pl.BlockSpec((_NW, _BPW), lambda: (0, 0))] * 2,
        out_shape=[jax.ShapeDtypeStruct((_NW, _BPW), jnp.float32)] * 2,
    )(d2)


def kernel(input, lut_weight, out_weight, out_bias):
    ids = input.astype(jnp.int32)
    pd = _project_table(lut_weight, out_weight, out_bias).reshape(-1)
    ids3 = ids.reshape(_NW, _BPW, _L).transpose(0, 2, 1)
    delta = _sc_gather_mean(pd, ids3)
    o0, o1 = _head(delta.reshape(_NW, _BPW))
    return jnp.stack([o0.reshape(_B), o1.reshape(_B)], axis=-1)
